# Initial kernel scaffold; baseline (speedup 1.0000x reference)
import jax
import jax.numpy as jnp
from jax.experimental import pallas as pl


def kernel(x, edge_index, W1l, b1, W1r, W2l, b2, W2r):
    raise NotImplementedError("write your pallas kernel here")


# speedup vs baseline: 4.5451x; 4.5451x over previous
import functools

import jax
import jax.numpy as jnp
from jax import lax
from jax.experimental import pallas as pl
from jax.experimental.pallas import tpu as pltpu
from jax.experimental.pallas import tpu_sc as plsc

N = 10000
E = 320000
D = 128
DH = D // 2

NC = 2
NS = 16
EPT = E // NS
K = 128
FULL = EPT // K
REM = EPT - FULL * K
NP = 10240
RPT = NP // NS
ZR = 128


def _make_sc_agg(with_counts):
    mesh = plsc.VectorSubcoreMesh(core_axis_name="c", subcore_axis_name="s")

    out_type = [jax.ShapeDtypeStruct((NC, NP, DH), jnp.float32)]
    if with_counts:
        out_type.append(jax.ShapeDtypeStruct((NP, 16), jnp.float32))

    scratch = [
        pltpu.VMEM((K,), jnp.int32),
        pltpu.VMEM((K,), jnp.int32),
        pltpu.VMEM((K, DH), jnp.float32),
        pltpu.VMEM((REM,), jnp.int32),
        pltpu.VMEM((REM,), jnp.int32),
        pltpu.VMEM((REM, DH), jnp.float32),
        pltpu.VMEM_SHARED((NP, DH), jnp.float32),
        pltpu.SemaphoreType.DMA,
    ]
    if with_counts:
        scratch += [
            pltpu.VMEM((K, 16), jnp.float32),
            pltpu.VMEM((REM, 16), jnp.float32),
            pltpu.VMEM((RPT, 16), jnp.float32),
            pltpu.VMEM_SHARED((NP, 16), jnp.float32),
        ]

    def body(*refs):
        if with_counts:
            (src, dst, feats, agg_out, cnt_out,
             idx_s, idx_d, rows, idx_s2, idx_d2, rows2,
             agg_sh, sem, ones, ones2, zcnt, cnt_sh) = refs
        else:
            (src, dst, feats, agg_out,
             idx_s, idx_d, rows, idx_s2, idx_d2, rows2,
             agg_sh, sem) = refs

        cid = lax.axis_index("c")
        sid = lax.axis_index("s")
        ebase = sid * EPT
        rbase = sid * RPT

        zvec = jnp.zeros((16,), jnp.float32)

        def zfill(i, c):
            for j in range(DH // 16):
                rows[i, pl.ds(j * 16, 16)] = zvec
            return c
        lax.fori_loop(0, ZR, zfill, 0)
        for r in range(RPT // ZR):
            pltpu.sync_copy(rows, agg_sh.at[pl.ds(rbase + r * ZR, ZR)])

        if with_counts:
            @pl.when(cid == 0)
            def _():
                ovec = jnp.ones((16,), jnp.float32)

                def cfill(i, c):
                    zcnt[i, :] = zvec
                    return c
                lax.fori_loop(0, RPT, cfill, 0)

                def ofill(i, c):
                    ones[i, :] = ovec
                    return c
                lax.fori_loop(0, K, ofill, 0)
                for i in range(REM):
                    ones2[i, :] = ovec
                pltpu.sync_copy(zcnt, cnt_sh.at[pl.ds(rbase, RPT)])

        plsc.subcore_barrier()

        def step(t, c):
            base = ebase + t * K
            pltpu.sync_copy(src.at[pl.ds(base, K)], idx_s)
            pltpu.sync_copy(dst.at[pl.ds(base, K)], idx_d)
            pltpu.async_copy(feats.at[cid].at[idx_s], rows, sem).wait()
            pltpu.sync_copy(rows, agg_sh.at[idx_d], add=True)
            if with_counts:
                @pl.when(cid == 0)
                def _():
                    pltpu.sync_copy(ones, cnt_sh.at[idx_d], add=True)
            return c
        lax.fori_loop(0, FULL, step, 0)

        if REM:
            base = ebase + FULL * K
            pltpu.sync_copy(src.at[pl.ds(base, REM)], idx_s2)
            pltpu.sync_copy(dst.at[pl.ds(base, REM)], idx_d2)
            pltpu.async_copy(feats.at[cid].at[idx_s2], rows2, sem).wait()
            pltpu.sync_copy(rows2, agg_sh.at[idx_d2], add=True)
            if with_counts:
                @pl.when(cid == 0)
                def _():
                    pltpu.sync_copy(ones2, cnt_sh.at[idx_d2], add=True)

        plsc.subcore_barrier()

        pltpu.sync_copy(agg_sh.at[pl.ds(rbase, RPT)],
                        agg_out.at[cid, pl.ds(rbase, RPT)])
        if with_counts:
            @pl.when(cid == 0)
            def _():
                pltpu.sync_copy(cnt_sh.at[pl.ds(rbase, RPT)],
                                cnt_out.at[pl.ds(rbase, RPT)])

    return pl.kernel(body, out_type=tuple(out_type), mesh=mesh,
                     scratch_types=scratch,
                     compiler_params=pltpu.CompilerParams(
                         use_tc_tiling_on_sc=False))


_sc_agg_counts = _make_sc_agg(True)
_sc_agg = _make_sc_agg(False)


BN = 1000


def _tc_body(relu, agg, cnt, xs, wl, wr, b, o, os):
    cs = cnt[:, 0]
    recip = 1.0 / jnp.maximum(cs, 1.0)
    aggm = jnp.concatenate([agg[0], agg[1]], axis=1) * recip[:, None]
    x = jnp.concatenate([xs[0], xs[1]], axis=1)
    dn = (((1,), (1,)), ((), ()))
    acc = lax.dot_general(aggm, wl[...], dn, preferred_element_type=jnp.float32)
    acc = acc + lax.dot_general(x, wr[...], dn,
                                preferred_element_type=jnp.float32)
    acc = acc + b[...]
    if relu:
        acc = jnp.maximum(acc, 0.0)
    o[...] = acc
    os[0] = acc[:, :DH]
    os[1] = acc[:, DH:]


def _tc_combine(agg, cnt, xs, Wl, Wr, b, relu):
    grid = N // BN
    return pl.pallas_call(
        functools.partial(_tc_body, relu),
        grid=(grid,),
        in_specs=[
            pl.BlockSpec((NC, BN, DH), lambda i: (0, i, 0)),
            pl.BlockSpec((BN, 16), lambda i: (i, 0)),
            pl.BlockSpec((NC, BN, DH), lambda i: (0, i, 0)),
            pl.BlockSpec((D, D), lambda i: (0, 0)),
            pl.BlockSpec((D, D), lambda i: (0, 0)),
            pl.BlockSpec((1, D), lambda i: (0, 0)),
        ],
        out_specs=[
            pl.BlockSpec((BN, D), lambda i: (i, 0)),
            pl.BlockSpec((NC, BN, DH), lambda i: (0, i, 0)),
        ],
        out_shape=[
            jax.ShapeDtypeStruct((N, D), jnp.float32),
            jax.ShapeDtypeStruct((NC, N, DH), jnp.float32),
        ],
    )(agg, cnt, xs, Wl, Wr, b)


@jax.jit
def kernel(x, edge_index, W1l, b1, W1r, W2l, b2, W2r):
    src = edge_index[0]
    dst = edge_index[1]
    b1r = b1.reshape(1, D)
    b2r = b2.reshape(1, D)
    xs = jnp.stack([x[:, :DH], x[:, DH:]])

    agg1, cnt = _sc_agg_counts(src, dst, xs)
    h, hs = _tc_combine(agg1, cnt, xs, W1l, W1r, b1r, relu=True)
    (agg2,) = _sc_agg(src, dst, hs)
    out, _ = _tc_combine(agg2, cnt, hs, W2l, W2r, b2r, relu=False)
    return out


# speedup vs baseline: 8.0666x; 1.7748x over previous
import functools

import jax
import jax.numpy as jnp
from jax import lax
from jax.experimental import pallas as pl
from jax.experimental.pallas import tpu as pltpu
from jax.experimental.pallas import tpu_sc as plsc

N = 10000
E = 320000
D = 128
DH = D // 2

NC = 2
NS = 16
K = 128
NCHUNK = E // K
B = 4
CPT = 156
NBLK = CPT // B
BK = B * K
NP = 10240
RPT = NP // NS


def _make_sc_agg(with_counts):
    mesh = plsc.VectorSubcoreMesh(core_axis_name="c", subcore_axis_name="s")

    out_type = [jax.ShapeDtypeStruct((NC, NP, DH), jnp.float32)]
    if with_counts:
        out_type.append(jax.ShapeDtypeStruct((NP, 16), jnp.float32))

    scratch = [
        pltpu.VMEM((B, K), jnp.int32),
        pltpu.VMEM((B, K), jnp.int32),
        pltpu.VMEM((BK, DH), jnp.float32),
        pltpu.VMEM_SHARED((NP, DH), jnp.float32),
        pltpu.SemaphoreType.DMA,
        pltpu.SemaphoreType.DMA,
    ]
    if with_counts:
        scratch += [
            pltpu.VMEM((K, 16), jnp.float32),
            pltpu.VMEM((64, 16), jnp.float32),
            pltpu.SemaphoreType.DMA,
            pltpu.VMEM_SHARED((NP, 16), jnp.float32),
        ]

    def body(*refs):
        if with_counts:
            (src, dst, feats, agg_out, cnt_out,
             sbuf, dbuf, rows, agg_sh, gsem, ssem,
             ones, zc, csem, cnt_sh) = refs
        else:
            (src, dst, feats, agg_out,
             sbuf, dbuf, rows, agg_sh, gsem, ssem) = refs

        cid = lax.axis_index("c")
        sid = lax.axis_index("s")
        c0 = sid * CPT
        rbase = sid * RPT

        zvec = jnp.zeros((16,), jnp.float32)

        def zfill(i, c):
            for j in range(DH // 16):
                rows[i, pl.ds(j * 16, 16)] = zvec
            return c
        lax.fori_loop(0, BK, zfill, 0)
        pltpu.sync_copy(rows, agg_sh.at[pl.ds(rbase, BK)])
        pltpu.sync_copy(rows.at[pl.ds(0, RPT - BK)],
                        agg_sh.at[pl.ds(rbase + BK, RPT - BK)])

        if with_counts:
            @pl.when(cid == 0)
            def _():
                ovec = jnp.ones((16,), jnp.float32)

                def czfill(i, c):
                    zc[i, :] = zvec
                    return c
                lax.fori_loop(0, 64, czfill, 0)

                def ofill(i, c):
                    ones[i, :] = ovec
                    return c
                lax.fori_loop(0, K, ofill, 0)
                for r in range(RPT // 64):
                    pltpu.sync_copy(zc, cnt_sh.at[pl.ds(rbase + r * 64, 64)])

        plsc.subcore_barrier()

        def do_block(cbase):
            pltpu.sync_copy(src.at[pl.ds(cbase, B)], sbuf)
            pltpu.sync_copy(dst.at[pl.ds(cbase, B)], dbuf)
            gd = [pltpu.async_copy(feats.at[cid].at[sbuf.at[j]],
                                   rows.at[pl.ds(j * K, K)], gsem)
                  for j in range(B)]
            for d in gd:
                d.wait()
            sd = [pltpu.async_copy(rows.at[pl.ds(j * K, K)],
                                   agg_sh.at[dbuf.at[j]], ssem, add=True)
                  for j in range(B)]
            if with_counts:
                @pl.when(cid == 0)
                def _():
                    cd = [pltpu.async_copy(ones, cnt_sh.at[dbuf.at[j]],
                                           csem, add=True)
                          for j in range(B)]
                    for d in cd:
                        d.wait()
            for d in sd:
                d.wait()

        def step(t, c):
            do_block(c0 + t * B)
            return c
        lax.fori_loop(0, NBLK, step, 0)

        @pl.when(sid == NS - 1)
        def _():
            do_block(NS * CPT)

        plsc.subcore_barrier()

        pltpu.sync_copy(agg_sh.at[pl.ds(rbase, RPT)],
                        agg_out.at[cid, pl.ds(rbase, RPT)])
        if with_counts:
            @pl.when(cid == 0)
            def _():
                pltpu.sync_copy(cnt_sh.at[pl.ds(rbase, RPT)],
                                cnt_out.at[pl.ds(rbase, RPT)])

    return pl.kernel(body, out_type=tuple(out_type), mesh=mesh,
                     scratch_types=scratch,
                     compiler_params=pltpu.CompilerParams(
                         use_tc_tiling_on_sc=False))


_sc_agg_counts = _make_sc_agg(True)
_sc_agg = _make_sc_agg(False)


BN = 1000


def _tc_body(relu, agg, cnt, xs, wl, wr, b, o, os):
    cs = cnt[:, 0]
    recip = 1.0 / jnp.maximum(cs, 1.0)
    aggm = jnp.concatenate([agg[0], agg[1]], axis=1) * recip[:, None]
    x = jnp.concatenate([xs[0], xs[1]], axis=1)
    dn = (((1,), (1,)), ((), ()))
    acc = lax.dot_general(aggm, wl[...], dn, preferred_element_type=jnp.float32)
    acc = acc + lax.dot_general(x, wr[...], dn,
                                preferred_element_type=jnp.float32)
    acc = acc + b[...]
    if relu:
        acc = jnp.maximum(acc, 0.0)
    o[...] = acc
    os[0] = acc[:, :DH]
    os[1] = acc[:, DH:]


def _tc_combine(agg, cnt, xs, Wl, Wr, b, relu):
    grid = N // BN
    return pl.pallas_call(
        functools.partial(_tc_body, relu),
        grid=(grid,),
        in_specs=[
            pl.BlockSpec((NC, BN, DH), lambda i: (0, i, 0)),
            pl.BlockSpec((BN, 16), lambda i: (i, 0)),
            pl.BlockSpec((NC, BN, DH), lambda i: (0, i, 0)),
            pl.BlockSpec((D, D), lambda i: (0, 0)),
            pl.BlockSpec((D, D), lambda i: (0, 0)),
            pl.BlockSpec((1, D), lambda i: (0, 0)),
        ],
        out_specs=[
            pl.BlockSpec((BN, D), lambda i: (i, 0)),
            pl.BlockSpec((NC, BN, DH), lambda i: (0, i, 0)),
        ],
        out_shape=[
            jax.ShapeDtypeStruct((N, D), jnp.float32),
            jax.ShapeDtypeStruct((NC, N, DH), jnp.float32),
        ],
    )(agg, cnt, xs, Wl, Wr, b)


@jax.jit
def kernel(x, edge_index, W1l, b1, W1r, W2l, b2, W2r):
    src = edge_index[0].reshape(NCHUNK, K)
    dst = edge_index[1].reshape(NCHUNK, K)
    b1r = b1.reshape(1, D)
    b2r = b2.reshape(1, D)
    xs = jnp.stack([x[:, :DH], x[:, DH:]])

    agg1, cnt = _sc_agg_counts(src, dst, xs)
    h, hs = _tc_combine(agg1, cnt, xs, W1l, W1r, b1r, relu=True)
    (agg2,) = _sc_agg(src, dst, hs)
    out, _ = _tc_combine(agg2, cnt, hs, W2l, W2r, b2r, relu=False)
    return out


# speedup vs baseline: 9.1129x; 1.1297x over previous
import functools

import jax
import jax.numpy as jnp
from jax import lax
from jax.experimental import pallas as pl
from jax.experimental.pallas import tpu as pltpu
from jax.experimental.pallas import tpu_sc as plsc

N = 10000
E = 320000
D = 128
DH = D // 2

NC = 2
NS = 16
K = 128
NCHUNK = E // K
B = 4
CPT = 156
NBLK = CPT // B
BK = B * K
NP = 10240
RPT = NP // NS


def _make_sc_agg(with_counts):
    mesh = plsc.VectorSubcoreMesh(core_axis_name="c", subcore_axis_name="s")

    out_type = [jax.ShapeDtypeStruct((NC, NP, DH), jnp.float32)]
    if with_counts:
        out_type.append(jax.ShapeDtypeStruct((NP, 16), jnp.float32))

    scratch = [
        pltpu.VMEM((B, K), jnp.int32),
        pltpu.VMEM((B, K), jnp.int32),
        pltpu.VMEM((BK, DH), jnp.float32),
        pltpu.VMEM_SHARED((NP, DH), jnp.float32),
        pltpu.SemaphoreType.DMA,
        pltpu.SemaphoreType.DMA,
    ]
    if with_counts:
        scratch += [
            pltpu.VMEM((K, 16), jnp.float32),
            pltpu.VMEM((64, 16), jnp.float32),
            pltpu.SemaphoreType.DMA,
            pltpu.VMEM_SHARED((NP, 16), jnp.float32),
        ]

    def body(*refs):
        if with_counts:
            (src, dst, feats, agg_out, cnt_out,
             sbuf, dbuf, rows, agg_sh, gsem, ssem,
             ones, zc, csem, cnt_sh) = refs
        else:
            (src, dst, feats, agg_out,
             sbuf, dbuf, rows, agg_sh, gsem, ssem) = refs

        cid = lax.axis_index("c")
        sid = lax.axis_index("s")
        c0 = sid * CPT
        rbase = sid * RPT

        zvec = jnp.zeros((16,), jnp.float32)

        def zfill(i, c):
            for j in range(DH // 16):
                rows[i, pl.ds(j * 16, 16)] = zvec
            return c
        lax.fori_loop(0, BK, zfill, 0)
        pltpu.sync_copy(rows, agg_sh.at[pl.ds(rbase, BK)])
        pltpu.sync_copy(rows.at[pl.ds(0, RPT - BK)],
                        agg_sh.at[pl.ds(rbase + BK, RPT - BK)])

        if with_counts:
            @pl.when(cid == 0)
            def _():
                ovec = jnp.ones((16,), jnp.float32)

                def czfill(i, c):
                    zc[i, :] = zvec
                    return c
                lax.fori_loop(0, 64, czfill, 0)

                def ofill(i, c):
                    ones[i, :] = ovec
                    return c
                lax.fori_loop(0, K, ofill, 0)
                for r in range(RPT // 64):
                    pltpu.sync_copy(zc, cnt_sh.at[pl.ds(rbase + r * 64, 64)])

        plsc.subcore_barrier()

        def do_block(cbase):
            pltpu.sync_copy(src.at[pl.ds(cbase, B)], sbuf)
            pltpu.sync_copy(dst.at[pl.ds(cbase, B)], dbuf)
            gd = [pltpu.async_copy(feats.at[cid].at[sbuf.at[j]],
                                   rows.at[pl.ds(j * K, K)], gsem)
                  for j in range(B)]
            sd = []
            for j in range(B):
                gd[j].wait()
                sd.append(pltpu.async_copy(rows.at[pl.ds(j * K, K)],
                                           agg_sh.at[dbuf.at[j]], ssem,
                                           add=True))
            if with_counts:
                @pl.when(cid == 0)
                def _():
                    cd = [pltpu.async_copy(ones, cnt_sh.at[dbuf.at[j]],
                                           csem, add=True)
                          for j in range(B)]
                    for d in cd:
                        d.wait()
            for d in sd:
                d.wait()

        def step(t, c):
            do_block(c0 + t * B)
            return c
        lax.fori_loop(0, NBLK, step, 0)

        @pl.when(sid == NS - 1)
        def _():
            do_block(NS * CPT)

        plsc.subcore_barrier()

        pltpu.sync_copy(agg_sh.at[pl.ds(rbase, RPT)],
                        agg_out.at[cid, pl.ds(rbase, RPT)])
        if with_counts:
            @pl.when(cid == 0)
            def _():
                pltpu.sync_copy(cnt_sh.at[pl.ds(rbase, RPT)],
                                cnt_out.at[pl.ds(rbase, RPT)])

    return pl.kernel(body, out_type=tuple(out_type), mesh=mesh,
                     scratch_types=scratch,
                     compiler_params=pltpu.CompilerParams(
                         use_tc_tiling_on_sc=False))


_sc_agg_counts = _make_sc_agg(True)
_sc_agg = _make_sc_agg(False)


BN = 1000


def _tc_body(relu, agg, cnt, xs, wl, wr, b, o, os):
    cs = cnt[:, 0]
    recip = 1.0 / jnp.maximum(cs, 1.0)
    aggm = jnp.concatenate([agg[0], agg[1]], axis=1) * recip[:, None]
    x = jnp.concatenate([xs[0], xs[1]], axis=1)
    dn = (((1,), (1,)), ((), ()))
    acc = lax.dot_general(aggm, wl[...], dn, preferred_element_type=jnp.float32)
    acc = acc + lax.dot_general(x, wr[...], dn,
                                preferred_element_type=jnp.float32)
    acc = acc + b[...]
    if relu:
        acc = jnp.maximum(acc, 0.0)
    o[...] = acc
    os[0] = acc[:, :DH]
    os[1] = acc[:, DH:]


def _tc_combine(agg, cnt, xs, Wl, Wr, b, relu):
    grid = N // BN
    return pl.pallas_call(
        functools.partial(_tc_body, relu),
        grid=(grid,),
        in_specs=[
            pl.BlockSpec((NC, BN, DH), lambda i: (0, i, 0)),
            pl.BlockSpec((BN, 16), lambda i: (i, 0)),
            pl.BlockSpec((NC, BN, DH), lambda i: (0, i, 0)),
            pl.BlockSpec((D, D), lambda i: (0, 0)),
            pl.BlockSpec((D, D), lambda i: (0, 0)),
            pl.BlockSpec((1, D), lambda i: (0, 0)),
        ],
        out_specs=[
            pl.BlockSpec((BN, D), lambda i: (i, 0)),
            pl.BlockSpec((NC, BN, DH), lambda i: (0, i, 0)),
        ],
        out_shape=[
            jax.ShapeDtypeStruct((N, D), jnp.float32),
            jax.ShapeDtypeStruct((NC, N, DH), jnp.float32),
        ],
    )(agg, cnt, xs, Wl, Wr, b)


@jax.jit
def kernel(x, edge_index, W1l, b1, W1r, W2l, b2, W2r):
    src = edge_index[0].reshape(NCHUNK, K)
    dst = edge_index[1].reshape(NCHUNK, K)
    b1r = b1.reshape(1, D)
    b2r = b2.reshape(1, D)
    xs = jnp.stack([x[:, :DH], x[:, DH:]])

    agg1, cnt = _sc_agg_counts(src, dst, xs)
    h, hs = _tc_combine(agg1, cnt, xs, W1l, W1r, b1r, relu=True)
    (agg2,) = _sc_agg(src, dst, hs)
    out, _ = _tc_combine(agg2, cnt, hs, W2l, W2r, b2r, relu=False)
    return out
